# baseline (speedup 1.0000x reference)
import functools

import jax
import jax.numpy as jnp
from jax import lax
from jax.experimental import pallas as pl
from jax.experimental.pallas import tpu as pltpu
from jax.experimental.pallas import tpu_sc as plsc

B, D, L = 16, 512, 4096

SC_B = 16
B0 = B - SC_B
DTC = 256
CG = (D - DTC) // 32
LB = 512
LANES = 16
NBUF = 8
CHUNK = 8 * LANES
LEN_PAD = 64

LBT = 512
NL = L // LBT


def _sc_body(x_hbm, len_hbm, out_hbm, buf, acc, out_stage, len_v, sems):
    c = lax.axis_index("c")
    s = lax.axis_index("s")
    wid = c * 16 + s
    ch0 = DTC + wid * CG

    pltpu.sync_copy(len_hbm, len_v.at[pl.ds(0, B)])
    lane = jnp.arange(LANES, dtype=jnp.int32)
    inf_v = jnp.full((LANES,), jnp.inf, dtype=jnp.float32)

    def nblocks_of(b):
        ln = len_v[pl.ds(b, LANES)][0]
        return (ln + (LB - 1)) // LB, ln

    def total_body(i, t):
        nb, _ = nblocks_of(B0 + i)
        return t + nb

    total_units = lax.fori_loop(0, SC_B, total_body, jnp.int32(0))

    def advance(st):
        b, blk, nb, ln = st
        nxt = blk + 1
        wrap = nxt == nb
        b2 = jnp.minimum(b + wrap.astype(jnp.int32), B - 1)
        blk2 = jnp.where(wrap, 0, nxt)
        nb2, ln2 = nblocks_of(b2)
        return (b2, blk2, jnp.where(wrap, nb2, nb), jnp.where(wrap, ln2, ln))

    def issue(u, st):
        b, blk, _, _ = st
        slot = u % NBUF
        pltpu.async_copy(
            x_hbm.at[b, pl.ds(ch0, CG), pl.ds(blk * LB, LB)],
            buf.at[slot],
            sems.at[slot],
        )

    def wait(u, st):
        b, blk, _, _ = st
        slot = u % NBUF
        pltpu.make_async_copy(
            x_hbm.at[b, pl.ds(ch0, CG), pl.ds(blk * LB, LB)],
            buf.at[slot],
            sems.at[slot],
        ).wait()

    def compute(u, st):
        b, blk, nb, ln = st
        slot = u % NBUF
        l0 = blk * LB
        navail = jnp.minimum(LB, ln - l0)
        n_chunks = navail // CHUNK
        rem = navail - n_chunks * CHUNK

        @pl.when(blk == 0)
        def _():
            def init_body(ch, carry):
                acc[ch] = inf_v
                return carry

            lax.fori_loop(0, CG, init_body, 0)

        def ch_body(ch, carry):
            a = acc[ch]

            def chunk_body(t, a2):
                base = t * CHUNK
                for jj in range(CHUNK // LANES):
                    v = buf[slot, ch, pl.ds(base + jj * LANES, LANES)]
                    a2 = jnp.minimum(a2, v)
                return a2

            a = lax.fori_loop(0, n_chunks, chunk_body, a)

            @pl.when(rem > 0)
            def _():
                a2 = a
                rbase = n_chunks * CHUNK
                for jj in range(CHUNK // LANES):
                    off = jj * LANES
                    v = buf[slot, ch, pl.ds(rbase + off, LANES)]
                    v = jnp.where(lane < rem - off, v, inf_v)
                    a2 = jnp.minimum(a2, v)
                acc[ch] = a2

            @pl.when(rem == 0)
            def _():
                acc[ch] = a

            return carry

        lax.fori_loop(0, CG, ch_body, 0)

        @pl.when(blk == nb - 1)
        def _():
            def pack_body(ch, res):
                m = acc[ch]
                for k in (8, 4, 2, 1):
                    perm = jnp.bitwise_xor(lane, k)
                    m = jnp.minimum(m, m.at[perm].get(mode="promise_in_bounds"))
                return jnp.where(lane == ch, m, res)

            out_stage[pl.ds((b - B0) * LANES, LANES)] = lax.fori_loop(
                0, CG, pack_body, inf_v)

    def pro_body(u, st):
        @pl.when(u < total_units)
        def _():
            issue(u, st)

        return advance(st)

    nb0, ln0 = nblocks_of(B0)
    st0 = (jnp.int32(B0), jnp.int32(0), nb0, ln0)
    ist = lax.fori_loop(0, NBUF - 1, pro_body, st0)

    def unit_body(u, carry):
        cst, ist = carry

        @pl.when(u + (NBUF - 1) < total_units)
        def _():
            issue(u + (NBUF - 1), ist)

        ist2 = advance(ist)
        wait(u, cst)
        compute(u, cst)
        return (advance(cst), ist2)

    lax.fori_loop(0, total_units, unit_body, (st0, ist))

    pltpu.sync_copy(out_stage, out_hbm.at[wid])


@functools.partial(
    pl.kernel,
    mesh=plsc.VectorSubcoreMesh(core_axis_name="c", subcore_axis_name="s"),
    out_type=jax.ShapeDtypeStruct((32, SC_B * LANES), jnp.float32),
    scratch_types=[
        pltpu.VMEM((NBUF, CG, LB), jnp.float32),
        pltpu.VMEM((CG, LANES), jnp.float32),
        pltpu.VMEM((SC_B * LANES,), jnp.float32),
        pltpu.VMEM((LEN_PAD,), jnp.int32),
        pltpu.SemaphoreType.DMA((NBUF,)),
    ],
)
def _sc_pool_min(x_hbm, len_hbm, out_hbm, buf, acc, out_stage, len_v, sems):
    _sc_body(x_hbm, len_hbm, out_hbm, buf, acc, out_stage, len_v, sems)


def _tc_index_x(cb, b, l, lens):
    nb = (lens[b] + (LBT - 1)) // LBT
    li = jnp.minimum(l, nb - 1)
    return (b, cb, li)


def _tc_index_o(cb, b, l, lens):
    return (b, 0, cb)


def _tc_body(lens_ref, x_ref, o_ref, acc_ref):
    cb = pl.program_id(0)
    b = pl.program_id(1)
    l = pl.program_id(2)
    ln = lens_ref[b]
    nb = (ln + (LBT - 1)) // LBT
    active = l < nb

    @pl.when(jnp.logical_and(active, l < nb - 1))
    def _():
        x = x_ref[...].reshape(DTC, LBT // 128, 128)
        m = jnp.min(x, axis=1)

        @pl.when(l == 0)
        def _():
            acc_ref[...] = m

        @pl.when(l > 0)
        def _():
            acc_ref[...] = jnp.minimum(acc_ref[...], m)

    @pl.when(l == nb - 1)
    def _():
        x = x_ref[...]
        pos = l * LBT + lax.broadcasted_iota(jnp.int32, (1, 1, LBT), 2)
        x = jnp.where(pos < ln, x, jnp.inf)
        m = jnp.min(x.reshape(DTC, LBT // 128, 128), axis=1)

        @pl.when(l > 0)
        def _():
            acc_ref[...] = jnp.minimum(acc_ref[...], m)

        @pl.when(l == 0)
        def _():
            acc_ref[...] = m

        o_ref[...] = jnp.min(acc_ref[...], axis=1).reshape(1, 1, DTC)


_tc_pool_min = pl.pallas_call(
    _tc_body,
    grid_spec=pltpu.PrefetchScalarGridSpec(
        num_scalar_prefetch=1,
        grid=(2, B, NL),
        in_specs=[pl.BlockSpec((1, DTC, LBT), _tc_index_x)],
        out_specs=pl.BlockSpec((1, 1, DTC), _tc_index_o),
        scratch_shapes=[pltpu.VMEM((DTC, 128), jnp.float32)],
    ),
    out_shape=jax.ShapeDtypeStruct((B, 1, D), jnp.float32),
    compiler_params=pltpu.CompilerParams(
        dimension_semantics=("arbitrary", "arbitrary", "arbitrary"),
    ),
)


def kernel(x0, x1, x2):
    del x1
    return _tc_pool_min(x2, x0).reshape(B, D)


# speedup vs baseline: 1.4983x; 1.4983x over previous
import functools

import jax
import jax.numpy as jnp
from jax import lax
from jax.experimental import pallas as pl
from jax.experimental.pallas import tpu as pltpu
from jax.experimental.pallas import tpu_sc as plsc

B, D, L = 16, 512, 4096

SC_B = 4
B0 = B - SC_B
CG = D // 32
LB = 512
LANES = 16
NBUF = 8
CHUNK = 8 * LANES
LEN_PAD = 64

LBT = 512
NL = L // LBT


def _sc_body(x_hbm, len_hbm, out_hbm, buf, acc, out_stage, len_v, sems):
    c = lax.axis_index("c")
    s = lax.axis_index("s")
    wid = c * 16 + s
    ch0 = wid * CG

    pltpu.sync_copy(len_hbm, len_v.at[pl.ds(0, B)])
    lane = jnp.arange(LANES, dtype=jnp.int32)
    inf_v = jnp.full((LANES,), jnp.inf, dtype=jnp.float32)

    def nblocks_of(b):
        ln = len_v[pl.ds(b, LANES)][0]
        return (ln + (LB - 1)) // LB, ln

    def total_body(i, t):
        nb, _ = nblocks_of(B0 + i)
        return t + nb

    total_units = lax.fori_loop(0, SC_B, total_body, jnp.int32(0))

    def advance(st):
        b, blk, nb, ln = st
        nxt = blk + 1
        wrap = nxt == nb
        b2 = jnp.minimum(b + wrap.astype(jnp.int32), B - 1)
        blk2 = jnp.where(wrap, 0, nxt)
        nb2, ln2 = nblocks_of(b2)
        return (b2, blk2, jnp.where(wrap, nb2, nb), jnp.where(wrap, ln2, ln))

    def issue(u, st):
        b, blk, _, _ = st
        slot = u % NBUF
        pltpu.async_copy(
            x_hbm.at[b, pl.ds(ch0, CG), pl.ds(blk * LB, LB)],
            buf.at[slot],
            sems.at[slot],
        )

    def wait(u, st):
        b, blk, _, _ = st
        slot = u % NBUF
        pltpu.make_async_copy(
            x_hbm.at[b, pl.ds(ch0, CG), pl.ds(blk * LB, LB)],
            buf.at[slot],
            sems.at[slot],
        ).wait()

    def compute(u, st):
        b, blk, nb, ln = st
        slot = u % NBUF
        l0 = blk * LB
        navail = jnp.minimum(LB, ln - l0)
        n_chunks = navail // CHUNK
        rem = navail - n_chunks * CHUNK

        @pl.when(blk == 0)
        def _():
            def init_body(ch, carry):
                acc[ch] = inf_v
                return carry

            lax.fori_loop(0, CG, init_body, 0)

        def ch_body(ch, carry):
            a = acc[ch]

            def chunk_body(t, a2):
                base = t * CHUNK
                for jj in range(CHUNK // LANES):
                    v = buf[slot, ch, pl.ds(base + jj * LANES, LANES)]
                    a2 = jnp.minimum(a2, v)
                return a2

            a = lax.fori_loop(0, n_chunks, chunk_body, a)

            @pl.when(rem > 0)
            def _():
                a2 = a
                rbase = n_chunks * CHUNK
                for jj in range(CHUNK // LANES):
                    off = jj * LANES
                    v = buf[slot, ch, pl.ds(rbase + off, LANES)]
                    v = jnp.where(lane < rem - off, v, inf_v)
                    a2 = jnp.minimum(a2, v)
                acc[ch] = a2

            @pl.when(rem == 0)
            def _():
                acc[ch] = a

            return carry

        lax.fori_loop(0, CG, ch_body, 0)

        @pl.when(blk == nb - 1)
        def _():
            def pack_body(ch, res):
                m = acc[ch]
                for k in (8, 4, 2, 1):
                    perm = jnp.bitwise_xor(lane, k)
                    m = jnp.minimum(m, m.at[perm].get(mode="promise_in_bounds"))
                return jnp.where(lane == ch, m, res)

            out_stage[pl.ds((b - B0) * LANES, LANES)] = lax.fori_loop(
                0, CG, pack_body, inf_v)

    def pro_body(u, st):
        @pl.when(u < total_units)
        def _():
            issue(u, st)

        return advance(st)

    nb0, ln0 = nblocks_of(B0)
    st0 = (jnp.int32(B0), jnp.int32(0), nb0, ln0)
    ist = lax.fori_loop(0, NBUF - 1, pro_body, st0)

    def unit_body(u, carry):
        cst, ist = carry

        @pl.when(u + (NBUF - 1) < total_units)
        def _():
            issue(u + (NBUF - 1), ist)

        ist2 = advance(ist)
        wait(u, cst)
        compute(u, cst)
        return (advance(cst), ist2)

    lax.fori_loop(0, total_units, unit_body, (st0, ist))

    pltpu.sync_copy(out_stage, out_hbm.at[wid])


@functools.partial(
    pl.kernel,
    mesh=plsc.VectorSubcoreMesh(core_axis_name="c", subcore_axis_name="s"),
    out_type=jax.ShapeDtypeStruct((32, SC_B * LANES), jnp.float32),
    scratch_types=[
        pltpu.VMEM((NBUF, CG, LB), jnp.float32),
        pltpu.VMEM((CG, LANES), jnp.float32),
        pltpu.VMEM((SC_B * LANES,), jnp.float32),
        pltpu.VMEM((LEN_PAD,), jnp.int32),
        pltpu.SemaphoreType.DMA((NBUF,)),
    ],
)
def _sc_pool_min(x_hbm, len_hbm, out_hbm, buf, acc, out_stage, len_v, sems):
    _sc_body(x_hbm, len_hbm, out_hbm, buf, acc, out_stage, len_v, sems)


def _tc_index_x(b, l, lens):
    nb = (lens[b] + (LBT - 1)) // LBT
    li = jnp.minimum(l, nb - 1)
    return (b, 0, li)


def _tc_index_o(b, l, lens):
    return (b, 0, 0)


def _tc_body(lens_ref, x_ref, o_ref, acc_ref):
    b = pl.program_id(0)
    l = pl.program_id(1)
    ln = lens_ref[b]
    nb = (ln + (LBT - 1)) // LBT

    @pl.when(l < nb - 1)
    def _():
        x = x_ref[...].reshape(D, LBT // 128, 128)
        m = jnp.min(x, axis=1)

        @pl.when(l == 0)
        def _():
            acc_ref[...] = m

        @pl.when(l > 0)
        def _():
            acc_ref[...] = jnp.minimum(acc_ref[...], m)

    @pl.when(l == nb - 1)
    def _():
        x = x_ref[...]
        pos = l * LBT + lax.broadcasted_iota(jnp.int32, (1, 1, LBT), 2)
        x = jnp.where(pos < ln, x, jnp.inf)
        m = jnp.min(x.reshape(D, LBT // 128, 128), axis=1)

        @pl.when(l > 0)
        def _():
            acc_ref[...] = jnp.minimum(acc_ref[...], m)

        @pl.when(l == 0)
        def _():
            acc_ref[...] = m

        o_ref[...] = jnp.min(acc_ref[...], axis=1).reshape(1, 1, D)


_tc_pool_min = pl.pallas_call(
    _tc_body,
    grid_spec=pltpu.PrefetchScalarGridSpec(
        num_scalar_prefetch=1,
        grid=(B0, NL),
        in_specs=[pl.BlockSpec((1, D, LBT), _tc_index_x)],
        out_specs=pl.BlockSpec((1, 1, D), _tc_index_o),
        scratch_shapes=[pltpu.VMEM((D, 128), jnp.float32)],
    ),
    out_shape=jax.ShapeDtypeStruct((B0, 1, D), jnp.float32),
    compiler_params=pltpu.CompilerParams(
        dimension_semantics=("arbitrary", "arbitrary"),
    ),
)


def kernel(x0, x1, x2):
    del x1
    sc_raw = _sc_pool_min(x0, x2)
    tc_out = _tc_pool_min(x2, x0).reshape(B0, D)
    sc_part = (
        sc_raw.reshape(32, SC_B, LANES)
        .transpose(1, 0, 2)
        .reshape(SC_B, D)
    )
    return jnp.concatenate([tc_out, sc_part], axis=0)
